# Initial kernel scaffold; baseline (speedup 1.0000x reference)
import functools

import jax
import jax.numpy as jnp
from jax import lax
from jax.experimental import pallas as pl
from jax.experimental.pallas import tpu as pltpu
from jax.experimental.pallas import tpu_sc as plsc

_NC = 2
_NS = 16
_NW = _NC * _NS

_B = 4096
_L = 50
_E = 64
_TOT = _B * _L
_C = 128
_ROWS_PER_W = _TOT // _NW
_CH = _ROWS_PER_W // _C


def _emb_body(src_idx, tgt_idx, src_tab, tgt_tab, src_out, tgt_out,
              idxs, idxt, buf0, buf1, s0, s1):
    wid = lax.axis_index("s") * _NC + lax.axis_index("c")
    chunk0 = wid * _CH
    row0 = wid * _ROWS_PER_W

    pltpu.sync_copy(src_idx.at[pl.ds(chunk0, _CH)], idxs)
    pltpu.sync_copy(tgt_idx.at[pl.ds(chunk0, _CH)], idxt)

    def run_table(tab, out, idxv):
        pltpu.async_copy(tab.at[idxv.at[0]], buf0, s0)

        @pl.loop(0, _CH, step=2)
        def _pair(i):
            pltpu.async_copy(tab.at[idxv.at[i + 1]], buf1, s1)
            pltpu.make_async_copy(tab.at[idxv.at[i]], buf0, s0).wait()
            pltpu.sync_copy(buf0, out.at[pl.ds(row0 + i * _C, _C)])

            @pl.when(i + 2 < _CH)
            def _():
                pltpu.async_copy(tab.at[idxv.at[i + 2]], buf0, s0)

            pltpu.make_async_copy(tab.at[idxv.at[i + 1]], buf1, s1).wait()
            pltpu.sync_copy(buf1, out.at[pl.ds(row0 + (i + 1) * _C, _C)])

    run_table(src_tab, src_out, idxs)
    run_table(tgt_tab, tgt_out, idxt)


@jax.jit
def _emb(src_idx2d, tgt_idx2d, src_table, tgt_table):
    mesh = plsc.VectorSubcoreMesh(core_axis_name="c", subcore_axis_name="s")
    out_type = [
        jax.ShapeDtypeStruct((_TOT, _E), jnp.float32),
        jax.ShapeDtypeStruct((_TOT, _E), jnp.float32),
    ]
    scratch = [
        pltpu.VMEM((_CH, _C), jnp.int32),
        pltpu.VMEM((_CH, _C), jnp.int32),
        pltpu.VMEM((_C, _E), jnp.float32),
        pltpu.VMEM((_C, _E), jnp.float32),
        pltpu.SemaphoreType.DMA,
        pltpu.SemaphoreType.DMA,
    ]
    fn = pl.kernel(_emb_body, out_type=out_type, mesh=mesh,
                   scratch_types=scratch)
    return fn(src_idx2d, tgt_idx2d, src_table, tgt_table)


def kernel(src_tokens, tgt_tokens, src_table, tgt_table):
    src_idx2d = src_tokens.astype(jnp.int32).reshape(_NW * _CH, _C)
    tgt_idx2d = tgt_tokens.astype(jnp.int32).reshape(_NW * _CH, _C)
    src_flat, tgt_flat = _emb(src_idx2d, tgt_idx2d, src_table, tgt_table)
    return (src_flat.reshape(_B, _L, _E), tgt_flat.reshape(_B, _L, _E))


# speedup vs baseline: 4.8536x; 4.8536x over previous
import functools

import jax
import jax.numpy as jnp
from jax import lax
from jax.experimental import pallas as pl
from jax.experimental.pallas import tpu as pltpu
from jax.experimental.pallas import tpu_sc as plsc

_NC = 2
_NS = 16
_NW = _NC * _NS

_B = 4096
_L = 50
_E = 64
_TOT = _B * _L
_C = 128
_ROWS_PER_W = _TOT // _NW
_CH = _ROWS_PER_W // _C


def _emb_body(src_idx, tgt_idx, src_tab, tgt_tab, src_out, tgt_out,
              idxs, idxt, buf0, buf1, s0, s1):
    wid = lax.axis_index("s") * _NC + lax.axis_index("c")
    row0 = wid * _ROWS_PER_W

    pltpu.sync_copy(src_idx.at[wid], idxs)
    pltpu.sync_copy(tgt_idx.at[wid], idxt)

    def run_table(tab, out, idxv):
        pltpu.async_copy(tab.at[idxv.at[0]], buf0, s0)

        @pl.loop(0, _CH, step=2)
        def _pair(i):
            pltpu.async_copy(tab.at[idxv.at[i + 1]], buf1, s1)
            pltpu.make_async_copy(tab.at[idxv.at[i]], buf0, s0).wait()
            pltpu.sync_copy(buf0, out.at[pl.ds(row0 + i * _C, _C)])

            @pl.when(i + 2 < _CH)
            def _():
                pltpu.async_copy(tab.at[idxv.at[i + 2]], buf0, s0)

            pltpu.make_async_copy(tab.at[idxv.at[i + 1]], buf1, s1).wait()
            pltpu.sync_copy(buf1, out.at[pl.ds(row0 + (i + 1) * _C, _C)])

    run_table(src_tab, src_out, idxs)
    run_table(tgt_tab, tgt_out, idxt)


@jax.jit
def _emb(src_idx2d, tgt_idx2d, src_table, tgt_table):
    mesh = plsc.VectorSubcoreMesh(core_axis_name="c", subcore_axis_name="s")
    out_type = [
        jax.ShapeDtypeStruct((_TOT, _E), jnp.float32),
        jax.ShapeDtypeStruct((_TOT, _E), jnp.float32),
    ]
    scratch = [
        pltpu.VMEM((_CH, _C), jnp.int32),
        pltpu.VMEM((_CH, _C), jnp.int32),
        pltpu.VMEM((_C, _E), jnp.float32),
        pltpu.VMEM((_C, _E), jnp.float32),
        pltpu.SemaphoreType.DMA,
        pltpu.SemaphoreType.DMA,
    ]
    fn = pl.kernel(_emb_body, out_type=out_type, mesh=mesh,
                   scratch_types=scratch,
                   compiler_params=pltpu.CompilerParams(
                       use_tc_tiling_on_sc=False))
    return fn(src_idx2d, tgt_idx2d, src_table, tgt_table)


def kernel(src_tokens, tgt_tokens, src_table, tgt_table):
    src_idx2d = src_tokens.astype(jnp.int32).reshape(_NW, _CH, _C)
    tgt_idx2d = tgt_tokens.astype(jnp.int32).reshape(_NW, _CH, _C)
    src_flat, tgt_flat = _emb(src_idx2d, tgt_idx2d, src_table, tgt_table)
    return (src_flat.reshape(_B, _L, _E), tgt_flat.reshape(_B, _L, _E))


# speedup vs baseline: 4.9482x; 1.0195x over previous
import functools

import jax
import jax.numpy as jnp
from jax import lax
from jax.experimental import pallas as pl
from jax.experimental.pallas import tpu as pltpu
from jax.experimental.pallas import tpu_sc as plsc

_NC = 2
_NS = 16
_NW = _NC * _NS

_B = 4096
_L = 50
_E = 64
_TOT = _B * _L
_C = 128
_ROWS_PER_W = _TOT // _NW
_CH = _ROWS_PER_W // _C


_G = 5
_NG = _CH // _G


def _emb_body(src_idx, tgt_idx, src_tab, tgt_tab, src_out, tgt_out,
              idxs, idxt, buf0, buf1, s0, s1, w0, w1):
    wid = lax.axis_index("s") * _NC + lax.axis_index("c")
    row0 = wid * _ROWS_PER_W

    pltpu.sync_copy(src_idx.at[wid], idxs)
    pltpu.sync_copy(tgt_idx.at[wid], idxt)

    def fire_group(tab, idxv, g, buf, sem):
        for k in range(_G):
            pltpu.async_copy(tab.at[idxv.at[g * _G + k]],
                             buf.at[pl.ds(k * _C, _C)], sem)

    def drain_group(tab, idxv, g, buf, sem):
        for k in range(_G):
            pltpu.make_async_copy(tab.at[idxv.at[g * _G + k]],
                                  buf.at[pl.ds(k * _C, _C)], sem).wait()

    def run_table(tab, out, idxv):
        def out_block(g):
            return out.at[pl.ds(row0 + g * _G * _C, _G * _C)]

        @pl.loop(0, _NG, step=2)
        def _pair(g):
            @pl.when(g >= 2)
            def _():
                pltpu.make_async_copy(buf0, out_block(g - 2), w0).wait()

            fire_group(tab, idxv, g, buf0, s0)

            @pl.when(g >= 1)
            def _():
                pltpu.make_async_copy(buf1, out_block(g - 1), w1).wait()

            fire_group(tab, idxv, g + 1, buf1, s1)
            drain_group(tab, idxv, g, buf0, s0)
            pltpu.async_copy(buf0, out_block(g), w0)
            drain_group(tab, idxv, g + 1, buf1, s1)
            pltpu.async_copy(buf1, out_block(g + 1), w1)

        pltpu.make_async_copy(buf0, out_block(_NG - 2), w0).wait()
        pltpu.make_async_copy(buf1, out_block(_NG - 1), w1).wait()

    run_table(src_tab, src_out, idxs)
    run_table(tgt_tab, tgt_out, idxt)


@jax.jit
def _emb(src_idx2d, tgt_idx2d, src_table, tgt_table):
    mesh = plsc.VectorSubcoreMesh(core_axis_name="c", subcore_axis_name="s")
    out_type = [
        jax.ShapeDtypeStruct((_TOT, _E), jnp.float32),
        jax.ShapeDtypeStruct((_TOT, _E), jnp.float32),
    ]
    scratch = [
        pltpu.VMEM((_CH, _C), jnp.int32),
        pltpu.VMEM((_CH, _C), jnp.int32),
        pltpu.VMEM((_G * _C, _E), jnp.float32),
        pltpu.VMEM((_G * _C, _E), jnp.float32),
        pltpu.SemaphoreType.DMA,
        pltpu.SemaphoreType.DMA,
        pltpu.SemaphoreType.DMA,
        pltpu.SemaphoreType.DMA,
    ]
    fn = pl.kernel(_emb_body, out_type=out_type, mesh=mesh,
                   scratch_types=scratch,
                   compiler_params=pltpu.CompilerParams(
                       use_tc_tiling_on_sc=False))
    return fn(src_idx2d, tgt_idx2d, src_table, tgt_table)


def kernel(src_tokens, tgt_tokens, src_table, tgt_table):
    src_idx2d = src_tokens.astype(jnp.int32).reshape(_NW, _CH, _C)
    tgt_idx2d = tgt_tokens.astype(jnp.int32).reshape(_NW, _CH, _C)
    src_flat, tgt_flat = _emb(src_idx2d, tgt_idx2d, src_table, tgt_table)
    return (src_flat.reshape(_B, _L, _E), tgt_flat.reshape(_B, _L, _E))
